# baseline (speedup 1.0000x reference)
import functools

import jax
import jax.numpy as jnp
from jax import lax
from jax.experimental import pallas as pl
from jax.experimental.pallas import tpu as pltpu
from jax.experimental.pallas import tpu_sc as plsc


@functools.lru_cache(maxsize=None)
def _make_gather(V, D, N, C):
    info = plsc.get_sparse_core_info()
    NC, NS = info.num_cores, info.num_subcores
    NW = NC * NS
    rows_per_w = N // NW
    assert N % NW == 0 and rows_per_w % C == 0 and C % 8 == 0
    n_chunks = rows_per_w // C
    mesh = plsc.VectorSubcoreMesh(core_axis_name="c", subcore_axis_name="s")

    @functools.partial(
        pl.kernel,
        mesh=mesh,
        out_type=jax.ShapeDtypeStruct((N, D), jnp.float32),
        scratch_types=[
            pltpu.VMEM((C,), jnp.int32),
            pltpu.VMEM((C, D), jnp.float32),
            pltpu.SemaphoreType.DMA,
        ],
        compiler_params=pltpu.CompilerParams(use_tc_tiling_on_sc=False),
    )
    def gather_kernel(table_hbm, idx_hbm, out_hbm, idx_v, rows_v, sem):
        wid = lax.axis_index("s") * NC + lax.axis_index("c")
        base = wid * rows_per_w

        def body(g, carry):
            off = base + g * C
            pltpu.sync_copy(idx_hbm.at[pl.ds(off, C)], idx_v)
            pltpu.async_copy(table_hbm.at[idx_v], rows_v, sem).wait()
            pltpu.sync_copy(rows_v, out_hbm.at[pl.ds(off, C)])
            return carry

        lax.fori_loop(0, n_chunks, body, 0)

    return gather_kernel


def kernel(X, embedding_matrix):
    B, H = X.shape
    V, D = embedding_matrix.shape
    idx = X.reshape(-1).astype(jnp.int32)
    out = _make_gather(V, D, B * H, 512)(embedding_matrix, idx)
    return out.reshape(B, H, D)


# speedup vs baseline: 1.0455x; 1.0455x over previous
import functools

import jax
import jax.numpy as jnp
from jax import lax
from jax.experimental import pallas as pl
from jax.experimental.pallas import tpu as pltpu
from jax.experimental.pallas import tpu_sc as plsc


@functools.lru_cache(maxsize=None)
def _make_gather(V, D, N, C):
    info = plsc.get_sparse_core_info()
    NC, NS = info.num_cores, info.num_subcores
    NW = NC * NS
    rows_per_w = N // NW
    assert N % NW == 0 and rows_per_w % C == 0 and C % 8 == 0
    n_chunks = rows_per_w // C
    assert n_chunks >= 4 and n_chunks % 2 == 0
    mesh = plsc.VectorSubcoreMesh(core_axis_name="c", subcore_axis_name="s")

    @functools.partial(
        pl.kernel,
        mesh=mesh,
        out_type=jax.ShapeDtypeStruct((N, D), jnp.float32),
        scratch_types=[
            pltpu.VMEM((n_chunks, C), jnp.int32),
            pltpu.VMEM((C, D), jnp.float32),
            pltpu.VMEM((C, D), jnp.float32),
            pltpu.SemaphoreType.DMA,
            pltpu.SemaphoreType.DMA,
            pltpu.SemaphoreType.DMA,
            pltpu.SemaphoreType.DMA,
        ],
        compiler_params=pltpu.CompilerParams(use_tc_tiling_on_sc=False),
    )
    def gather_kernel(table_hbm, idx_hbm, out_hbm, idx_all, rows0, rows1,
                      sg0, sg1, so0, so1):
        wid = lax.axis_index("s") * NC + lax.axis_index("c")
        base = wid * rows_per_w
        rows = (rows0, rows1)
        sg = (sg0, sg1)
        so = (so0, so1)

        pltpu.sync_copy(idx_hbm.at[wid], idx_all)

        def start_gather(g, b):
            pltpu.async_copy(table_hbm.at[idx_all.at[g]], rows[b], sg[b])

        def wait_gather(g, b):
            pltpu.make_async_copy(table_hbm.at[idx_all.at[g]], rows[b],
                                  sg[b]).wait()

        def start_out(g, b):
            pltpu.async_copy(rows[b], out_hbm.at[pl.ds(base + g * C, C)],
                             so[b])

        def wait_out(g, b):
            pltpu.make_async_copy(rows[b], out_hbm.at[pl.ds(base + g * C, C)],
                                  so[b]).wait()

        start_gather(0, 0)
        start_gather(1, 1)
        wait_gather(0, 0)
        start_out(0, 0)

        def body(i, carry):
            g = 1 + 2 * i
            wait_out(g - 1, 0)
            start_gather(g + 1, 0)
            wait_gather(g, 1)
            start_out(g, 1)
            wait_out(g, 1)
            start_gather(g + 2, 1)
            wait_gather(g + 1, 0)
            start_out(g + 1, 0)
            return carry

        lax.fori_loop(0, (n_chunks - 2) // 2, body, 0)

        g_last = n_chunks - 1
        wait_gather(g_last, 1)
        start_out(g_last, 1)
        wait_out(g_last - 1, 0)
        wait_out(g_last, 1)

    return gather_kernel


def kernel(X, embedding_matrix):
    B, H = X.shape
    V, D = embedding_matrix.shape
    C = 512
    info = plsc.get_sparse_core_info()
    NW = info.num_cores * info.num_subcores
    N = B * H
    idx = X.reshape(NW, (N // NW) // C, C).astype(jnp.int32)
    out = _make_gather(V, D, N, C)(embedding_matrix, idx)
    return out.reshape(B, H, D)
